# Initial kernel scaffold; baseline (speedup 1.0000x reference)
import jax
import jax.numpy as jnp
from jax.experimental import pallas as pl


def kernel(output, mask, ind, target):
    raise NotImplementedError("write your pallas kernel here")


# speedup vs baseline: 2.8396x; 2.8396x over previous
import jax
import jax.numpy as jnp
from jax import lax
from jax.experimental import pallas as pl
from jax.experimental.pallas import tpu as pltpu
from jax.experimental.pallas import tpu_sc as plsc

B, C, H, W, K = 16, 2, 512, 512, 256
HW = H * W
L = 16
NWORK = 16
PAIRS = B * K
PW = PAIRS // NWORK
EW = PW * C
ROWS_PER_DMA = 128
NPASS = EW // ROWS_PER_DMA
CHUNKS_PER_PASS = ROWS_PER_DMA // L


def _sc_body(table, ind_flat, mask_flat, target_flat, out_hbm,
             ind_v, mask_v, tgt_v, idx_v, col_v, rows_v, term_v, accs_v,
             part_v, out_v, shared, sem):
    cid = lax.axis_index("c")
    sid = lax.axis_index("s")

    @pl.when(cid == 0)
    def _work():
        base_pair = sid * PW
        pltpu.sync_copy(ind_flat.at[pl.ds(base_pair, PW)], ind_v)
        pltpu.sync_copy(mask_flat.at[pl.ds(base_pair, PW)], mask_v)
        pltpu.sync_copy(target_flat.at[pl.ds(base_pair * C, EW)], tgt_v)
        b = base_pair // K
        for i in range(EW // L):
            e = lax.iota(jnp.int32, L) + i * L
            pair = e >> 1
            ch = e & 1
            p = plsc.load_gather(ind_v, [pair])
            idx_v[pl.ds(i * L, L)] = (b * C + ch) * H + (p >> 9)
            col_v[pl.ds(i * L, L)] = p & (W - 1)
        for ps in range(NPASS):
            pltpu.async_copy(
                table.at[idx_v.at[pl.ds(ps * ROWS_PER_DMA, ROWS_PER_DMA)]],
                rows_v, sem).wait()
            for i in range(CHUNKS_PER_PASS):
                off = ps * ROWS_PER_DMA + i * L
                e = lax.iota(jnp.int32, L) + off
                pair = e >> 1
                e_loc = lax.iota(jnp.int32, L) + i * L
                col = col_v[pl.ds(off, L)]
                vals = plsc.load_gather(rows_v, [e_loc, col])
                m = plsc.load_gather(mask_v, [pair])
                t = tgt_v[pl.ds(off, L)]
                term_v[pl.ds(off, L)] = jnp.abs(vals * m - t * m)
        acc = jnp.zeros((L,), jnp.float32)
        macc = jnp.zeros((L,), jnp.float32)
        for i in range(EW // L):
            e = lax.iota(jnp.int32, L) + i * L
            acc = acc + term_v[pl.ds(i * L, L)]
            macc = macc + plsc.load_gather(mask_v, [e >> 1])
        accs_v[pl.ds(0, L)] = acc
        accs_v[pl.ds(L, L)] = macc
        pltpu.sync_copy(accs_v, shared.at[pl.ds(sid * 2 * L, 2 * L)])

    plsc.subcore_barrier()

    @pl.when((cid == 0) & (sid == 0))
    def _finish():
        pltpu.sync_copy(shared, part_v)
        a = jnp.zeros((L,), jnp.float32)
        ma = jnp.zeros((L,), jnp.float32)
        for w_ in range(NWORK):
            a = a + part_v[pl.ds(w_ * 2 * L, L)]
            ma = ma + part_v[pl.ds(w_ * 2 * L + L, L)]
        lsum = jnp.sum(a)
        msum = jnp.sum(ma)
        lv = jnp.full((L,), lsum, jnp.float32)
        mv = jnp.full((L,), msum, jnp.float32)
        out_v[...] = lv / (mv + jnp.float32(1e-4))
        pltpu.sync_copy(out_v, out_hbm)


_launch = pl.kernel(
    _sc_body,
    out_type=jax.ShapeDtypeStruct((L,), jnp.float32),
    mesh=plsc.VectorSubcoreMesh(core_axis_name="c", subcore_axis_name="s"),
    compiler_params=pltpu.CompilerParams(
        needs_layout_passes=False, use_tc_tiling_on_sc=True),
    scratch_types=[
        pltpu.VMEM((PW,), jnp.int32),
        pltpu.VMEM((PW,), jnp.float32),
        pltpu.VMEM((EW,), jnp.float32),
        pltpu.VMEM((EW,), jnp.int32),
        pltpu.VMEM((EW,), jnp.int32),
        pltpu.VMEM((ROWS_PER_DMA, W), jnp.float32),
        pltpu.VMEM((EW,), jnp.float32),
        pltpu.VMEM((2 * L,), jnp.float32),
        pltpu.VMEM((NWORK * 2 * L,), jnp.float32),
        pltpu.VMEM((L,), jnp.float32),
        pltpu.VMEM_SHARED((NWORK * 2 * L,), jnp.float32),
        pltpu.SemaphoreType.DMA,
    ],
)


def kernel(output, mask, ind, target):
    table = output.reshape(B * C * H, W)
    ind_flat = ind.reshape(PAIRS)
    mask_flat = mask.reshape(PAIRS)
    target_flat = target.reshape(PAIRS * C)
    out = _launch(table, ind_flat, mask_flat, target_flat)
    return out[0]


# speedup vs baseline: 3.1963x; 1.1256x over previous
import jax
import jax.numpy as jnp
from jax import lax
from jax.experimental import pallas as pl
from jax.experimental.pallas import tpu as pltpu
from jax.experimental.pallas import tpu_sc as plsc

B, C, H, W, K = 16, 2, 512, 512, 256
HW = H * W
L = 16
NC = 2
NS = 16
NWORK = NC * NS
PAIRS = B * K
PW = PAIRS // NWORK
EW = PW * C
ROWS_PER_DMA = 64
NPASS = EW // ROWS_PER_DMA
CHUNKS_PER_PASS = ROWS_PER_DMA // L


def _sc_body(table, ind_flat, mask_flat, target_flat, part_hbm,
             ind_v, mask_v, tgt_v, idx_v, col_v, rows0_v, rows1_v, term_v,
             accs_v, red_v, shared, sem0, sem1):
    cid = lax.axis_index("c")
    sid = lax.axis_index("s")
    wid = cid * NS + sid
    base_pair = wid * PW
    pltpu.sync_copy(ind_flat.at[pl.ds(base_pair, PW)], ind_v)
    pltpu.sync_copy(mask_flat.at[pl.ds(base_pair, PW)], mask_v)
    pltpu.sync_copy(target_flat.at[pl.ds(base_pair * C, EW)], tgt_v)
    b = base_pair // K
    for i in range(EW // L):
        e = lax.iota(jnp.int32, L) + i * L
        pair = e >> 1
        ch = e & 1
        p = plsc.load_gather(ind_v, [pair])
        idx_v[pl.ds(i * L, L)] = (b * C + ch) * H + (p >> 9)
        col_v[pl.ds(i * L, L)] = p & (W - 1)

    rows_bufs = (rows0_v, rows1_v)
    sems = (sem0, sem1)

    def fire(ps):
        return pltpu.async_copy(
            table.at[idx_v.at[pl.ds(ps * ROWS_PER_DMA, ROWS_PER_DMA)]],
            rows_bufs[ps % 2], sems[ps % 2])

    pending = fire(0)
    for ps in range(NPASS):
        pending.wait()
        if ps + 1 < NPASS:
            pending = fire(ps + 1)
        rows_v = rows_bufs[ps % 2]
        for i in range(CHUNKS_PER_PASS):
            off = ps * ROWS_PER_DMA + i * L
            pair = (lax.iota(jnp.int32, L) + off) >> 1
            e_loc = lax.iota(jnp.int32, L) + i * L
            col = col_v[pl.ds(off, L)]
            vals = plsc.load_gather(rows_v, [e_loc, col])
            m = plsc.load_gather(mask_v, [pair])
            t = tgt_v[pl.ds(off, L)]
            term_v[pl.ds(off, L)] = jnp.abs(vals * m - t * m)
    acc = jnp.zeros((L,), jnp.float32)
    macc = jnp.zeros((L,), jnp.float32)
    for i in range(EW // L):
        e = lax.iota(jnp.int32, L) + i * L
        acc = acc + term_v[pl.ds(i * L, L)]
        macc = macc + plsc.load_gather(mask_v, [e >> 1])
    accs_v[pl.ds(0, L)] = acc
    accs_v[pl.ds(L, L)] = macc
    pltpu.sync_copy(accs_v, shared.at[pl.ds(sid * 2 * L, 2 * L)])

    plsc.subcore_barrier()

    @pl.when(sid == 0)
    def _reduce_core():
        pltpu.sync_copy(shared, red_v)
        a = jnp.zeros((L,), jnp.float32)
        ma = jnp.zeros((L,), jnp.float32)
        for w_ in range(NS):
            a = a + red_v[pl.ds(w_ * 2 * L, L)]
            ma = ma + red_v[pl.ds(w_ * 2 * L + L, L)]
        accs_v[pl.ds(0, L)] = a
        accs_v[pl.ds(L, L)] = ma
        pltpu.sync_copy(accs_v, part_hbm.at[cid])


_sc_launch = pl.kernel(
    _sc_body,
    out_type=jax.ShapeDtypeStruct((NC, 2 * L), jnp.float32),
    mesh=plsc.VectorSubcoreMesh(core_axis_name="c", subcore_axis_name="s"),
    compiler_params=pltpu.CompilerParams(
        needs_layout_passes=False, use_tc_tiling_on_sc=True),
    scratch_types=[
        pltpu.VMEM((PW,), jnp.int32),
        pltpu.VMEM((PW,), jnp.float32),
        pltpu.VMEM((EW,), jnp.float32),
        pltpu.VMEM((EW,), jnp.int32),
        pltpu.VMEM((EW,), jnp.int32),
        pltpu.VMEM((ROWS_PER_DMA, W), jnp.float32),
        pltpu.VMEM((ROWS_PER_DMA, W), jnp.float32),
        pltpu.VMEM((EW,), jnp.float32),
        pltpu.VMEM((2 * L,), jnp.float32),
        pltpu.VMEM((NS * 2 * L,), jnp.float32),
        pltpu.VMEM_SHARED((NS * 2 * L,), jnp.float32),
        pltpu.SemaphoreType.DMA,
        pltpu.SemaphoreType.DMA,
    ],
)


def _combine_body(part_ref, out_ref):
    lsum = jnp.sum(part_ref[:, 0:L])
    msum = jnp.sum(part_ref[:, L:2 * L])
    out_ref[...] = jnp.full((1, 1), lsum / (msum + jnp.float32(1e-4)),
                            jnp.float32)


_combine = pl.pallas_call(
    _combine_body,
    out_shape=jax.ShapeDtypeStruct((1, 1), jnp.float32),
)


def kernel(output, mask, ind, target):
    table = output.reshape(B * C * H, W)
    ind_flat = ind.reshape(PAIRS)
    mask_flat = mask.reshape(PAIRS)
    target_flat = target.reshape(PAIRS * C)
    part = _sc_launch(table, ind_flat, mask_flat, target_flat)
    return _combine(part)[0, 0]
